# baseline (speedup 1.0000x reference)
import functools

import jax
import jax.numpy as jnp
from jax import lax
from jax.experimental import pallas as pl
from jax.experimental.pallas import tpu as pltpu
from jax.experimental.pallas import tpu_sc as plsc

_NUM_WORKERS = 32
_GROUP = 8
_NBUF = 2
_TC_BATCH_BLOCK = 512
_HIST_SPLIT = 26


def _make_gather(batch, hist_c, embed_dim, dtype):
    mesh = plsc.VectorSubcoreMesh(core_axis_name="c", subcore_axis_name="s")
    per_w = batch // _NUM_WORKERS
    ch = _GROUP * hist_c
    steps = per_w // _GROUP
    n_c = batch * hist_c

    @functools.partial(
        pl.kernel,
        out_type=jax.ShapeDtypeStruct((n_c, embed_dim), dtype),
        mesh=mesh,
        compiler_params=pltpu.CompilerParams(use_tc_tiling_on_sc=False),
        scratch_types=[
            pltpu.VMEM((1, per_w * hist_c), jnp.int32),
            pltpu.VMEM((_NBUF, ch, embed_dim), jnp.float32),
            pltpu.SemaphoreType.DMA,
            pltpu.SemaphoreType.DMA,
            pltpu.SemaphoreType.DMA,
            pltpu.SemaphoreType.DMA,
        ],
    )
    def gather_kernel(table_hbm, idx_hbm, out_hbm, idx_v, rows_v, g0, g1, w0, w1):
        gsem = [g0, g1]
        wsem = [w0, w1]
        wid = lax.axis_index("s") * 2 + lax.axis_index("c")
        b0 = wid * per_w

        pltpu.sync_copy(
            idx_hbm.at[0, pl.ds(b0 * hist_c, per_w * hist_c)], idx_v.at[0]
        )

        def issue_gather(k, slot):
            pltpu.async_copy(
                table_hbm.at[idx_v.at[0, pl.ds(k * ch, ch)]],
                rows_v.at[slot],
                gsem[slot],
            )

        def wait_gather(slot):
            pltpu.make_async_copy(
                table_hbm.at[pl.ds(0, ch)], rows_v.at[slot], gsem[slot]
            ).wait()

        def issue_writes(k, slot):
            pltpu.async_copy(
                rows_v.at[slot],
                out_hbm.at[pl.ds((b0 + k * _GROUP) * hist_c, ch)],
                wsem[slot],
            )

        def drain_writes(slot):
            pltpu.make_async_copy(
                rows_v.at[slot], out_hbm.at[pl.ds(0, ch)], wsem[slot]
            ).wait()

        issue_gather(0, 0)

        @pl.loop(0, steps, step=_NBUF)
        def _(t):
            for b in range(_NBUF):
                k = t + b
                nslot = (b + 1) % _NBUF

                @pl.when(k + 1 < steps)
                def _prefetch():
                    @pl.when(k + 1 >= _NBUF)
                    def _drain():
                        drain_writes(nslot)

                    issue_gather(k + 1, nslot)

                wait_gather(b)
                issue_writes(k, b)

        for b in range(_NBUF):
            drain_writes(b)

    return gather_kernel


def kernel(style_idx, table):
    batch, hist = style_idx.shape
    num_rows, embed_dim = table.shape
    row = hist * embed_dim
    bblk = _TC_BATCH_BLOCK
    chunks = [(0, _HIST_SPLIT), (_HIST_SPLIT, hist)]

    xt = None
    for ci, (h_lo, h_hi) in enumerate(chunks):
        hist_c = h_hi - h_lo
        rpb = hist_c * embed_dim // 128
        r_base = h_lo * embed_dim // 128

        idx_c = style_idx[:, h_lo:h_hi].reshape(1, batch * hist_c)
        idx_c = idx_c.astype(jnp.int32)
        g = _make_gather(batch, hist_c, embed_dim, table.dtype)(table, idx_c)

        in2d = g.reshape(batch * rpb, 128)

        def transpose_body(*refs, rpb=rpb, r_base=r_base):
            in_ref, out_ref = refs[0], refs[-1]
            r = pl.program_id(1)
            out_ref[...] = in_ref[pl.Slice(r, bblk, rpb), :].T

        operands = [in2d] if xt is None else [in2d, xt]
        in_specs = [
            pl.BlockSpec((bblk * rpb, 128), lambda j, r: (j, 0)),
        ]
        if xt is not None:
            in_specs.append(pl.BlockSpec(memory_space=pltpu.MemorySpace.HBM))
        xt = pl.pallas_call(
            transpose_body,
            out_shape=jax.ShapeDtypeStruct((row, batch), table.dtype),
            grid=(batch // bblk, rpb),
            in_specs=in_specs,
            out_specs=pl.BlockSpec(
                (128, bblk), lambda j, r, rb=r_base: (rb + r, j)
            ),
            input_output_aliases={} if len(operands) == 1 else {1: 0},
            compiler_params=pltpu.CompilerParams(
                dimension_semantics=("parallel", "arbitrary")
            ),
        )(*operands)

    x3 = xt.reshape(hist, embed_dim, batch)
    return jnp.transpose(x3, (2, 0, 1))
